# baseline (speedup 1.0000x reference)
import jax
import jax.numpy as jnp
from jax.experimental import pallas as pl
from jax.experimental.pallas import tpu as pltpu

_TT = 512


def _moe_body(x_ref, wg_ref, bg_ref, wef_ref,
              out_ref, scores_ref, xs_ref):
    D = x_ref.shape[1]
    E = wg_ref.shape[1]

    x = x_ref[...]
    logits = jnp.dot(x, wg_ref[...], preferred_element_type=jnp.float32)
    logits = logits + bg_ref[...]
    m = jnp.max(logits, axis=-1, keepdims=True)
    e = jnp.exp(logits - m)
    s = e / jnp.sum(e, axis=-1, keepdims=True)
    scores_ref[...] = s

    col = jax.lax.broadcasted_iota(jnp.int32, s.shape, 1)
    for k in range(E):
        s_k = jnp.sum(jnp.where(col == k, s, 0.0), axis=1, keepdims=True)
        xs_ref[:, k * D:(k + 1) * D] = (x * s_k).astype(jnp.bfloat16)

    s128 = jnp.concatenate([s] * (128 // E), axis=1).astype(jnp.bfloat16)
    xs_ref[:, E * D:] = s128
    out_ref[...] = jnp.dot(xs_ref[...], wef_ref[...],
                           preferred_element_type=jnp.float32)


def kernel(x, Wg, bg, We, be):
    T, D = x.shape
    E = Wg.shape[1]
    bep = jnp.zeros((128, D), jnp.float32).at[:E].set(be)
    wef = jnp.concatenate([We.reshape(E * D, D), bep], axis=0).astype(jnp.bfloat16)

    out, scores = pl.pallas_call(
        _moe_body,
        grid=(T // _TT,),
        in_specs=[
            pl.BlockSpec((_TT, D), lambda i: (i, 0)),
            pl.BlockSpec((D, E), lambda i: (0, 0)),
            pl.BlockSpec((1, E), lambda i: (0, 0)),
            pl.BlockSpec((E * D + 128, D), lambda i: (0, 0)),
        ],
        out_specs=[
            pl.BlockSpec((_TT, D), lambda i: (i, 0)),
            pl.BlockSpec((_TT, E), lambda i: (i, 0)),
        ],
        out_shape=[
            jax.ShapeDtypeStruct((T, D), jnp.float32),
            jax.ShapeDtypeStruct((T, E), jnp.float32),
        ],
        scratch_shapes=[pltpu.VMEM((_TT, E * D + 128), jnp.bfloat16)],
        compiler_params=pltpu.CompilerParams(
            dimension_semantics=("arbitrary",),
        ),
    )(x, Wg, bg.reshape(1, E), wef)
    return out, scores


# speedup vs baseline: 1.0224x; 1.0224x over previous
import jax
import jax.numpy as jnp
from jax.experimental import pallas as pl
from jax.experimental.pallas import tpu as pltpu

_TT = 1024


def _moe_body(x_ref, wg_ref, bg_ref, wef_ref, bep_ref,
              out_ref, scores_ref, xs_ref):
    D = x_ref.shape[1]
    E = wg_ref.shape[1]

    x = x_ref[...]
    logits = jnp.dot(x, wg_ref[...], preferred_element_type=jnp.float32)
    logits = logits + bg_ref[...]
    m = jnp.max(logits, axis=-1, keepdims=True)
    e = jnp.exp(logits - m)
    s = e / jnp.sum(e, axis=-1, keepdims=True)
    scores_ref[...] = s

    col = jax.lax.broadcasted_iota(jnp.int32, s.shape, 1)
    for k in range(E):
        s_k = jnp.sum(jnp.where(col == k, s, 0.0), axis=1, keepdims=True)
        xs_ref[:, k * D:(k + 1) * D] = (x * s_k).astype(jnp.bfloat16)

    s128 = jnp.concatenate([s] * (128 // E), axis=1).astype(jnp.bfloat16)
    out_ref[...] = (
        jnp.dot(xs_ref[...], wef_ref[...], preferred_element_type=jnp.float32)
        + jnp.dot(s128, bep_ref[...], preferred_element_type=jnp.float32)
    )


def kernel(x, Wg, bg, We, be):
    T, D = x.shape
    E = Wg.shape[1]
    wef = We.reshape(E * D, D).astype(jnp.bfloat16)
    bep = jnp.zeros((128, D), jnp.bfloat16).at[:E].set(be.astype(jnp.bfloat16))

    out, scores = pl.pallas_call(
        _moe_body,
        grid=(T // _TT,),
        in_specs=[
            pl.BlockSpec((_TT, D), lambda i: (i, 0)),
            pl.BlockSpec((D, E), lambda i: (0, 0)),
            pl.BlockSpec((1, E), lambda i: (0, 0)),
            pl.BlockSpec((E * D, D), lambda i: (0, 0)),
            pl.BlockSpec((128, D), lambda i: (0, 0)),
        ],
        out_specs=[
            pl.BlockSpec((_TT, D), lambda i: (i, 0)),
            pl.BlockSpec((_TT, E), lambda i: (i, 0)),
        ],
        out_shape=[
            jax.ShapeDtypeStruct((T, D), jnp.float32),
            jax.ShapeDtypeStruct((T, E), jnp.float32),
        ],
        scratch_shapes=[pltpu.VMEM((_TT, E * D), jnp.bfloat16)],
        compiler_params=pltpu.CompilerParams(
            dimension_semantics=("arbitrary",),
        ),
    )(x, Wg, bg.reshape(1, E), wef, bep)
    return out, scores
